# speedup vs baseline: 1.2357x; 1.2357x over previous
import functools

import jax
import jax.numpy as jnp
from jax import lax
from jax.experimental import pallas as pl
from jax.experimental.pallas import tpu as pltpu
from jax.experimental.pallas import tpu_sc as plsc

F32 = jnp.float32
I32 = jnp.int32

NA, NP_, NT = 10000, 50000, 5000
D = 128
O = 64
E_RAW = 500000
E_PAD = 503808
B = 128
PER_TILE = E_PAD // 16
NBAT = PER_TILE // B
CH_DATA = 12160
CH_TOT = 12288

NAP = 12160
NPP = 60800
NTP = 12160


def _make_agg(nch):
    n_out = nch * CH_DATA
    mesh = plsc.VectorSubcoreMesh(core_axis_name="c", subcore_axis_name="s")

    @functools.partial(
        pl.kernel,
        mesh=mesh,
        out_type=(jax.ShapeDtypeStruct((n_out, D), F32),
                  jax.ShapeDtypeStruct((n_out,), F32)),
        scratch_types=[
            pltpu.VMEM_SHARED((CH_TOT, D), F32),
            pltpu.VMEM_SHARED((CH_TOT,), F32),
            pltpu.VMEM((B,), I32),
            pltpu.VMEM((B,), I32),
            pltpu.VMEM((B,), I32),
            pltpu.VMEM((B, D), F32),
            pltpu.VMEM((B,), F32),
            pltpu.VMEM((768,), F32),
            pltpu.SemaphoreType.DMA,
        ],
    )
    def agg(src_hbm, dst_hbm, x_hbm, s_hbm, c_hbm,
            acc, cacc, srcb, dstb, lidxb, rowsb, onesb, cbufb, sem):
        cid = lax.axis_index("c")
        sid = lax.axis_index("s")
        zero16 = jnp.zeros((16,), F32)
        one16 = jnp.ones((16,), F32)
        for k in range(B // 16):
            onesb[pl.ds(16 * k, 16)] = one16
        for k in range(768 // 16):
            cbufb[pl.ds(16 * k, 16)] = zero16

        def zero_rows(i, carry):
            for k in range(D // 16):
                rowsb[i, pl.ds(16 * k, 16)] = zero16
            return carry

        for ch in range(nch):
            @pl.when(cid == (ch % 2))
            def _chunk(ch=ch):
                base = ch * CH_DATA
                lax.fori_loop(0, B, zero_rows, 0)
                for z in range(CH_TOT // 16 // B):
                    pltpu.sync_copy(rowsb, acc.at[pl.ds(sid * 768 + z * B, B)])
                pltpu.sync_copy(cbufb, cacc.at[pl.ds(sid * 768, 768)])
                plsc.subcore_barrier()

                def edge_batch(i, carry):
                    off = sid * PER_TILE + i * B
                    pltpu.sync_copy(src_hbm.at[pl.ds(off, B)], srcb)
                    pltpu.sync_copy(dst_hbm.at[pl.ds(off, B)], dstb)
                    for k in range(B // 16):
                        d = dstb[pl.ds(16 * k, 16)]
                        lv = d - base
                        oob = (lv < 0) | (lv >= CH_DATA)
                        garb = CH_DATA + (d & 127)
                        lidxb[pl.ds(16 * k, 16)] = jnp.where(oob, garb, lv)
                    pltpu.async_copy(x_hbm.at[srcb], rowsb, sem).wait()
                    pltpu.sync_copy(rowsb, acc.at[lidxb], add=True)
                    pltpu.sync_copy(onesb, cacc.at[lidxb], add=True)
                    return carry

                lax.fori_loop(0, NBAT, edge_batch, 0)
                plsc.subcore_barrier()

                row0 = sid * 760
                for z in range(5):
                    pltpu.sync_copy(acc.at[pl.ds(row0 + z * B, B)], rowsb)
                    pltpu.sync_copy(rowsb, s_hbm.at[pl.ds(base + row0 + z * B, B)])
                pltpu.sync_copy(acc.at[pl.ds(row0 + 640, 120)], rowsb.at[pl.ds(0, 120)])
                pltpu.sync_copy(rowsb.at[pl.ds(0, 120)], s_hbm.at[pl.ds(base + row0 + 640, 120)])
                pltpu.sync_copy(cacc.at[pl.ds(row0, 760)], cbufb.at[pl.ds(0, 760)])
                pltpu.sync_copy(cbufb.at[pl.ds(0, 760)], c_hbm.at[pl.ds(base + row0, 760)])
                for k in range(768 // 16):
                    cbufb[pl.ds(16 * k, 16)] = zero16
                plsc.subcore_barrier()

    return agg


def _make_combine(n_rows, n_rel):
    R = 320

    def body(*refs):
        x_ref = refs[2 * n_rel]
        wl = refs[2 * n_rel + 1: 2 * n_rel + 1 + n_rel]
        wr = refs[3 * n_rel + 1]
        b = refs[3 * n_rel + 2]
        o = refs[-1]
        acc = jnp.dot(x_ref[...], wr[...], preferred_element_type=F32) + b[...]
        for r in range(n_rel):
            s = refs[2 * r][...]
            c = refs[2 * r + 1][...]
            mean = s / jnp.maximum(c, 1.0)
            acc = acc + jnp.dot(mean, wl[r][...], preferred_element_type=F32)
        o[...] = jnp.maximum(acc, 0.0)

    in_specs = []
    for _ in range(n_rel):
        in_specs.append(pl.BlockSpec((R, D), lambda i: (i, 0)))
        in_specs.append(pl.BlockSpec((R, 1), lambda i: (i, 0)))
    in_specs.append(pl.BlockSpec((R, D), lambda i: (i, 0)))
    for _ in range(n_rel):
        in_specs.append(pl.BlockSpec((D, D), lambda i: (0, 0)))
    in_specs.append(pl.BlockSpec((D, D), lambda i: (0, 0)))
    in_specs.append(pl.BlockSpec((1, D), lambda i: (0, 0)))
    return pl.pallas_call(
        body,
        grid=(n_rows // R,),
        in_specs=in_specs,
        out_specs=pl.BlockSpec((R, D), lambda i: (i, 0)),
        out_shape=jax.ShapeDtypeStruct((n_rows, D), F32),
    )


def _make_head(n_rows):
    R = 320

    def body(hm, hg, lw, lb, gw, gb, l2w, l2b, o):
        logits = jnp.dot(hm[...], lw[...], preferred_element_type=F32) + lb[...]
        ll1 = jnp.dot(hg[...], gw[...], preferred_element_type=F32) + gb[...]
        temp = jnp.dot(ll1, l2w[...], preferred_element_type=F32) + l2b[...]
        o[...] = logits / temp

    in_specs = [
        pl.BlockSpec((R, D), lambda i: (i, 0)),
        pl.BlockSpec((R, D), lambda i: (i, 0)),
        pl.BlockSpec((D, O), lambda i: (0, 0)),
        pl.BlockSpec((1, O), lambda i: (0, 0)),
        pl.BlockSpec((D, O), lambda i: (0, 0)),
        pl.BlockSpec((1, O), lambda i: (0, 0)),
        pl.BlockSpec((O, 1), lambda i: (0, 0)),
        pl.BlockSpec((1, 1), lambda i: (0, 0)),
    ]
    return pl.pallas_call(
        body,
        grid=(n_rows // R,),
        in_specs=in_specs,
        out_specs=pl.BlockSpec((R, O), lambda i: (i, 0)),
        out_shape=jax.ShapeDtypeStruct((n_rows, O), F32),
    )


def kernel(x_author, x_paper, x_term, edge_ap, edge_pa, edge_pt, edge_tp, params):
    xa = jnp.pad(x_author, ((0, NAP - NA), (0, 0)))
    xp = jnp.pad(x_paper, ((0, NPP - NP_), (0, 0)))
    xt = jnp.pad(x_term, ((0, NTP - NT), (0, 0)))

    npad = E_PAD - E_RAW

    def prep_edges(e):
        src = jnp.concatenate([e[0], jnp.arange(npad, dtype=I32) % 997])
        dst = jnp.concatenate([e[1], jnp.full((npad,), -1, I32)])
        return src, dst

    eap = prep_edges(edge_ap)
    epa = prep_edges(edge_pa)
    ept = prep_edges(edge_pt)
    etp = prep_edges(edge_tp)

    agg1 = _make_agg(1)
    agg5 = _make_agg(5)

    comb_a = _make_combine(NAP, 1)
    comb_p = _make_combine(NPP, 2)
    comb_t = _make_combine(NTP, 1)
    head = _make_head(NAP)

    def aggregate(xd):
        return {
            "ap": agg5(eap[0], eap[1], xd["author"]),
            "pa": agg1(epa[0], epa[1], xd["paper"]),
            "pt": agg1(ept[0], ept[1], xd["paper"]),
            "tp": agg5(etp[0], etp[1], xd["term"]),
        }

    def hetero(aggs, xd, lp):
        sa, ca = aggs["pa"]
        out_a = comb_a(sa, ca.reshape(-1, 1), xd["author"],
                       lp["pa"]["Wl"], lp["pa"]["Wr"], lp["pa"]["bl"].reshape(1, D))
        s1, c1 = aggs["ap"]
        s2, c2 = aggs["tp"]
        out_p = comb_p(s1, c1.reshape(-1, 1), s2, c2.reshape(-1, 1), xd["paper"],
                       lp["ap"]["Wl"], lp["tp"]["Wl"],
                       lp["ap"]["Wr"] + lp["tp"]["Wr"],
                       (lp["ap"]["bl"] + lp["tp"]["bl"]).reshape(1, D))
        st, ct = aggs["pt"]
        out_t = comb_t(st, ct.reshape(-1, 1), xd["term"],
                       lp["pt"]["Wl"], lp["pt"]["Wr"], lp["pt"]["bl"].reshape(1, D))
        return {"author": out_a, "paper": out_p, "term": out_t}

    m = params["model"]
    g = params["gts"]
    cv = params["convs"]

    xd0 = {"author": xa, "paper": xp, "term": xt}
    p1 = aggregate(xd0)
    h1m = hetero(p1, xd0, m["layers"][0])
    h1c = hetero(p1, xd0, cv[0])
    p2 = aggregate(h1m)
    h2m = hetero(p2, h1m, m["layers"][1])
    p3 = aggregate(h1c)
    h2c = hetero(p3, h1c, cv[1])
    p4 = aggregate(h2m)
    h1g = hetero(p4, h2m, g["layers"][0])
    p5 = aggregate(h1g)
    h2g = hetero(p5, h1g, g["layers"][1])

    out0 = head(h2m["author"], h2g["author"],
                m["lin_W"], m["lin_b"].reshape(1, O),
                g["lin_W"], g["lin_b"].reshape(1, O),
                params["lin2_W"], params["lin2_b"].reshape(1, 1))
    return (out0[:NA], h2c["author"][:NA], h2c["paper"][:NP_], h2c["term"][:NT])


# speedup vs baseline: 2.4221x; 1.9602x over previous
import functools

import jax
import jax.numpy as jnp
from jax import lax
from jax.experimental import pallas as pl
from jax.experimental.pallas import tpu as pltpu
from jax.experimental.pallas import tpu_sc as plsc

F32 = jnp.float32
I32 = jnp.int32

NA, NP_, NT = 10000, 50000, 5000
D = 128
O = 64
E_RAW = 500000
E_PAD = 503808
B = 128
PER_TILE = E_PAD // 16
NBAT = PER_TILE // B

NAP = 12288
NPP = 58240
NTP = 5120


def _steps(n):
    out = []
    for s in (128, 64, 32, 16, 8):
        while n >= s:
            out.append(s)
            n -= s
    assert n == 0
    return out


def _make_agg(nch, chrows, parity):
    n_out = nch * chrows
    ch_tot = chrows + 128
    zr = ch_tot // 16
    dr = chrows // 16
    mesh = plsc.VectorSubcoreMesh(core_axis_name="c", subcore_axis_name="s")

    @functools.partial(
        pl.kernel,
        mesh=mesh,
        out_type=(jax.ShapeDtypeStruct((n_out, D), F32),
                  jax.ShapeDtypeStruct((n_out,), F32)),
        scratch_types=[
            pltpu.VMEM_SHARED((ch_tot, D), F32),
            pltpu.VMEM_SHARED((ch_tot,), F32),
            pltpu.VMEM((2, B), I32),
            pltpu.VMEM((2, B), I32),
            pltpu.VMEM((B,), I32),
            pltpu.VMEM((B,), I32),
            pltpu.VMEM((B,), I32),
            pltpu.VMEM((B,), I32),
            pltpu.VMEM((B, D), F32),
            pltpu.VMEM((B, D), F32),
            pltpu.VMEM((B,), F32),
            pltpu.VMEM((768,), F32),
            pltpu.SemaphoreType.DMA,
            pltpu.SemaphoreType.DMA,
            pltpu.SemaphoreType.DMA,
            pltpu.SemaphoreType.DMA,
            pltpu.SemaphoreType.DMA,
            pltpu.SemaphoreType.DMA,
        ],
    )
    def agg(edges_hbm, x_hbm, s_hbm, c_hbm,
            acc, cacc, eb0, eb1, sb0, sb1, lb0, lb1, rb0, rb1,
            onesb, cbufb, gs0, gs1, ss0, ss1, cs0, cs1):
        cid = lax.axis_index("c")
        sid = lax.axis_index("s")
        eb = (eb0, eb1)
        sb = (sb0, sb1)
        lb = (lb0, lb1)
        rb = (rb0, rb1)
        gs = (gs0, gs1)
        ss = (ss0, ss1)
        cs = (cs0, cs1)
        zero16 = jnp.zeros((16,), F32)
        one16 = jnp.ones((16,), F32)
        for k in range(B // 16):
            onesb[pl.ds(16 * k, 16)] = one16
        for k in range(768 // 16):
            cbufb[pl.ds(16 * k, 16)] = zero16

        def zero_rows(i, carry):
            for k in range(D // 16):
                rb0[i, pl.ds(16 * k, 16)] = zero16
            return carry

        def prep(j, u, base):
            off = sid * PER_TILE + j * B
            pltpu.sync_copy(edges_hbm.at[:, pl.ds(off, B)], eb[u])
            for k in range(B // 16):
                sl = pl.ds(16 * k, 16)
                sb[u][sl] = eb[u][0, sl]
                d = eb[u][1, sl]
                lv = d - base
                oob = (lv < 0) | (lv >= chrows)
                garb = chrows + (d & 127)
                lb[u][sl] = jnp.where(oob, garb, lv)
            pltpu.async_copy(x_hbm.at[sb[u]], rb[u], gs[u])

        def consume(u):
            pltpu.make_async_copy(x_hbm.at[sb[u]], rb[u], gs[u]).wait()
            pltpu.async_copy(rb[u], acc.at[lb[u]], ss[u], add=True)
            pltpu.async_copy(onesb, cacc.at[lb[u]], cs[u], add=True)

        def drain(u):
            pltpu.make_async_copy(rb[u], acc.at[lb[u]], ss[u]).wait()
            pltpu.make_async_copy(onesb, cacc.at[lb[u]], cs[u]).wait()

        for ch in range(nch):
            @pl.when(cid == ((ch + parity) % 2))
            def _chunk(ch=ch):
                base = ch * chrows
                lax.fori_loop(0, B, zero_rows, 0)
                r0 = sid * zr
                for st in _steps(zr):
                    pltpu.sync_copy(rb0.at[pl.ds(0, st)], acc.at[pl.ds(r0, st)])
                    r0 += st
                pltpu.sync_copy(cbufb.at[pl.ds(0, zr)], cacc.at[pl.ds(sid * zr, zr)])
                plsc.subcore_barrier()

                prep(0, 0, base)
                prep(1, 1, base)

                def pipe(i2, carry):
                    j = 2 * i2
                    consume(0)
                    consume(1)
                    drain(0)
                    prep(j + 2, 0, base)
                    drain(1)
                    prep(j + 3, 1, base)
                    return carry

                lax.fori_loop(0, (NBAT - 2) // 2, pipe, 0)
                consume(0)
                consume(1)
                drain(0)
                drain(1)
                plsc.subcore_barrier()

                r0 = sid * dr
                for st in _steps(dr):
                    pltpu.sync_copy(acc.at[pl.ds(r0, st)], rb0.at[pl.ds(0, st)])
                    pltpu.sync_copy(rb0.at[pl.ds(0, st)], s_hbm.at[pl.ds(base + r0, st)])
                    r0 += st
                pltpu.sync_copy(cacc.at[pl.ds(sid * dr, dr)], cbufb.at[pl.ds(0, dr)])
                pltpu.sync_copy(cbufb.at[pl.ds(0, dr)], c_hbm.at[pl.ds(base + sid * dr, dr)])
                for k in range(768 // 16):
                    cbufb[pl.ds(16 * k, 16)] = zero16
                plsc.subcore_barrier()

    return agg


def _make_combine(n_rows, n_rel):
    R = 128

    def body(*refs):
        x_ref = refs[2 * n_rel]
        wl = refs[2 * n_rel + 1: 2 * n_rel + 1 + n_rel]
        wr = refs[3 * n_rel + 1]
        b = refs[3 * n_rel + 2]
        o = refs[-1]
        acc = jnp.dot(x_ref[...], wr[...], preferred_element_type=F32) + b[...]
        for r in range(n_rel):
            s = refs[2 * r][...]
            c = refs[2 * r + 1][...]
            mean = s / jnp.maximum(c, 1.0)
            acc = acc + jnp.dot(mean, wl[r][...], preferred_element_type=F32)
        o[...] = jnp.maximum(acc, 0.0)

    in_specs = []
    for _ in range(n_rel):
        in_specs.append(pl.BlockSpec((R, D), lambda i: (i, 0)))
        in_specs.append(pl.BlockSpec((R, 1), lambda i: (i, 0)))
    in_specs.append(pl.BlockSpec((R, D), lambda i: (i, 0)))
    for _ in range(n_rel):
        in_specs.append(pl.BlockSpec((D, D), lambda i: (0, 0)))
    in_specs.append(pl.BlockSpec((D, D), lambda i: (0, 0)))
    in_specs.append(pl.BlockSpec((1, D), lambda i: (0, 0)))
    return pl.pallas_call(
        body,
        grid=(n_rows // R,),
        in_specs=in_specs,
        out_specs=pl.BlockSpec((R, D), lambda i: (i, 0)),
        out_shape=jax.ShapeDtypeStruct((n_rows, D), F32),
    )


def _make_head(n_rows):
    R = 128

    def body(hm, hg, lw, lb, gw, gb, l2w, l2b, o):
        logits = jnp.dot(hm[...], lw[...], preferred_element_type=F32) + lb[...]
        ll1 = jnp.dot(hg[...], gw[...], preferred_element_type=F32) + gb[...]
        temp = jnp.dot(ll1, l2w[...], preferred_element_type=F32) + l2b[...]
        o[...] = logits / temp

    in_specs = [
        pl.BlockSpec((R, D), lambda i: (i, 0)),
        pl.BlockSpec((R, D), lambda i: (i, 0)),
        pl.BlockSpec((D, O), lambda i: (0, 0)),
        pl.BlockSpec((1, O), lambda i: (0, 0)),
        pl.BlockSpec((D, O), lambda i: (0, 0)),
        pl.BlockSpec((1, O), lambda i: (0, 0)),
        pl.BlockSpec((O, 1), lambda i: (0, 0)),
        pl.BlockSpec((1, 1), lambda i: (0, 0)),
    ]
    return pl.pallas_call(
        body,
        grid=(n_rows // R,),
        in_specs=in_specs,
        out_specs=pl.BlockSpec((R, O), lambda i: (i, 0)),
        out_shape=jax.ShapeDtypeStruct((n_rows, O), F32),
    )


def kernel(x_author, x_paper, x_term, edge_ap, edge_pa, edge_pt, edge_tp, params):
    xa = jnp.pad(x_author, ((0, NAP - NA), (0, 0)))
    xp = jnp.pad(x_paper, ((0, NPP - NP_), (0, 0)))
    xt = jnp.pad(x_term, ((0, NTP - NT), (0, 0)))

    npad = E_PAD - E_RAW

    def prep_edges(e):
        src = jnp.concatenate([e[0], jnp.arange(npad, dtype=I32) % 997])
        dst = jnp.concatenate([e[1], jnp.full((npad,), -1, I32)])
        return jnp.stack([src, dst])

    eap = prep_edges(edge_ap)
    epa = prep_edges(edge_pa)
    ept = prep_edges(edge_pt)
    etp = prep_edges(edge_tp)

    agg_ap = _make_agg(5, 11648, 0)
    agg_tp = _make_agg(5, 11648, 1)
    agg_pa = _make_agg(2, 6144, 0)
    agg_pt = _make_agg(2, 2560, 0)

    comb_a = _make_combine(NAP, 1)
    comb_p = _make_combine(NPP, 2)
    comb_t = _make_combine(NTP, 1)
    head = _make_head(NAP)

    def aggregate(xd):
        return {
            "ap": agg_ap(eap, xd["author"]),
            "pa": agg_pa(epa, xd["paper"]),
            "pt": agg_pt(ept, xd["paper"]),
            "tp": agg_tp(etp, xd["term"]),
        }

    def hetero(aggs, xd, lp):
        sa, ca = aggs["pa"]
        out_a = comb_a(sa, ca.reshape(-1, 1), xd["author"],
                       lp["pa"]["Wl"], lp["pa"]["Wr"], lp["pa"]["bl"].reshape(1, D))
        s1, c1 = aggs["ap"]
        s2, c2 = aggs["tp"]
        out_p = comb_p(s1, c1.reshape(-1, 1), s2, c2.reshape(-1, 1), xd["paper"],
                       lp["ap"]["Wl"], lp["tp"]["Wl"],
                       lp["ap"]["Wr"] + lp["tp"]["Wr"],
                       (lp["ap"]["bl"] + lp["tp"]["bl"]).reshape(1, D))
        st, ct = aggs["pt"]
        out_t = comb_t(st, ct.reshape(-1, 1), xd["term"],
                       lp["pt"]["Wl"], lp["pt"]["Wr"], lp["pt"]["bl"].reshape(1, D))
        return {"author": out_a, "paper": out_p, "term": out_t}

    m = params["model"]
    g = params["gts"]
    cv = params["convs"]

    xd0 = {"author": xa, "paper": xp, "term": xt}
    p1 = aggregate(xd0)
    h1m = hetero(p1, xd0, m["layers"][0])
    h1c = hetero(p1, xd0, cv[0])
    p2 = aggregate(h1m)
    h2m = hetero(p2, h1m, m["layers"][1])
    p3 = aggregate(h1c)
    h2c = hetero(p3, h1c, cv[1])
    p4 = aggregate(h2m)
    h1g = hetero(p4, h2m, g["layers"][0])
    p5 = aggregate(h1g)
    h2g = hetero(p5, h1g, g["layers"][1])

    out0 = head(h2m["author"], h2g["author"],
                m["lin_W"], m["lin_b"].reshape(1, O),
                g["lin_W"], g["lin_b"].reshape(1, O),
                params["lin2_W"], params["lin2_b"].reshape(1, 1))
    return (out0[:NA], h2c["author"][:NA], h2c["paper"][:NP_], h2c["term"][:NT])


# speedup vs baseline: 2.4532x; 1.0128x over previous
import functools

import jax
import jax.numpy as jnp
from jax import lax
from jax.experimental import pallas as pl
from jax.experimental.pallas import tpu as pltpu
from jax.experimental.pallas import tpu_sc as plsc

F32 = jnp.float32
I32 = jnp.int32

NA, NP_, NT = 10000, 50000, 5000
D = 128
O = 64
E_RAW = 500000
E_PAD = 503808
B = 128
PER_TILE = E_PAD // 16
NBAT = PER_TILE // B

NAP = 12288
NPP = 58240
NTP = 5120


def _steps(n):
    out = []
    for s in (128, 64, 32, 16, 8):
        while n >= s:
            out.append(s)
            n -= s
    assert n == 0
    return out


def _make_agg(nch, chrows, parity, with_counts=True):
    n_out = nch * chrows
    ch_tot = chrows + 128
    zr = ch_tot // 16
    dr = chrows // 16
    mesh = plsc.VectorSubcoreMesh(core_axis_name="c", subcore_axis_name="s")

    out_type = [jax.ShapeDtypeStruct((n_out, D), F32)]
    scratch = [
        pltpu.VMEM_SHARED((ch_tot, D), F32),
        pltpu.VMEM((2, B), I32),
        pltpu.VMEM((2, B), I32),
        pltpu.VMEM((B,), I32),
        pltpu.VMEM((B,), I32),
        pltpu.VMEM((B,), I32),
        pltpu.VMEM((B,), I32),
        pltpu.VMEM((B, D), F32),
        pltpu.VMEM((B, D), F32),
        pltpu.SemaphoreType.DMA,
        pltpu.SemaphoreType.DMA,
        pltpu.SemaphoreType.DMA,
        pltpu.SemaphoreType.DMA,
    ]
    if with_counts:
        out_type.append(jax.ShapeDtypeStruct((n_out,), F32))
        scratch += [
            pltpu.VMEM_SHARED((ch_tot,), F32),
            pltpu.VMEM((B,), F32),
            pltpu.VMEM((768,), F32),
            pltpu.SemaphoreType.DMA,
            pltpu.SemaphoreType.DMA,
        ]

    @functools.partial(pl.kernel, mesh=mesh,
                       out_type=tuple(out_type) if with_counts else out_type[0],
                       scratch_types=scratch)
    def agg(edges_hbm, x_hbm, s_hbm, *rest):
        if with_counts:
            (c_hbm, acc, eb0, eb1, sb0, sb1, lb0, lb1, rb0, rb1,
             gs0, gs1, ss0, ss1, cacc, onesb, cbufb, cs0, cs1) = rest
            cs = (cs0, cs1)
        else:
            (acc, eb0, eb1, sb0, sb1, lb0, lb1, rb0, rb1,
             gs0, gs1, ss0, ss1) = rest
        cid = lax.axis_index("c")
        sid = lax.axis_index("s")
        eb = (eb0, eb1)
        sb = (sb0, sb1)
        lb = (lb0, lb1)
        rb = (rb0, rb1)
        gs = (gs0, gs1)
        ss = (ss0, ss1)
        zero16 = jnp.zeros((16,), F32)
        if with_counts:
            one16 = jnp.ones((16,), F32)
            for k in range(B // 16):
                onesb[pl.ds(16 * k, 16)] = one16
            for k in range(768 // 16):
                cbufb[pl.ds(16 * k, 16)] = zero16

        def zero_rows(i, carry):
            for k in range(D // 16):
                rb0[i, pl.ds(16 * k, 16)] = zero16
            return carry

        def prep(j, u, base):
            off = sid * PER_TILE + j * B
            pltpu.sync_copy(edges_hbm.at[:, pl.ds(off, B)], eb[u])
            for k in range(B // 16):
                sl = pl.ds(16 * k, 16)
                sb[u][sl] = eb[u][0, sl]
                d = eb[u][1, sl]
                lv = d - base
                oob = (lv < 0) | (lv >= chrows)
                garb = chrows + (d & 127)
                lb[u][sl] = jnp.where(oob, garb, lv)
            pltpu.async_copy(x_hbm.at[sb[u]], rb[u], gs[u])

        def consume(u):
            pltpu.make_async_copy(x_hbm.at[sb[u]], rb[u], gs[u]).wait()
            pltpu.async_copy(rb[u], acc.at[lb[u]], ss[u], add=True)
            if with_counts:
                pltpu.async_copy(onesb, cacc.at[lb[u]], cs[u], add=True)

        def drain(u):
            pltpu.make_async_copy(rb[u], acc.at[lb[u]], ss[u]).wait()
            if with_counts:
                pltpu.make_async_copy(onesb, cacc.at[lb[u]], cs[u]).wait()

        for ch in range(nch):
            @pl.when(cid == ((ch + parity) % 2))
            def _chunk(ch=ch):
                base = ch * chrows
                lax.fori_loop(0, B, zero_rows, 0)
                r0 = sid * zr
                for st in _steps(zr):
                    pltpu.sync_copy(rb0.at[pl.ds(0, st)], acc.at[pl.ds(r0, st)])
                    r0 += st
                if with_counts:
                    pltpu.sync_copy(cbufb.at[pl.ds(0, zr)],
                                    cacc.at[pl.ds(sid * zr, zr)])
                plsc.subcore_barrier()

                prep(0, 0, base)
                prep(1, 1, base)

                def pipe(i2, carry):
                    j = 2 * i2
                    consume(0)
                    consume(1)
                    drain(0)
                    prep(j + 2, 0, base)
                    drain(1)
                    prep(j + 3, 1, base)
                    return carry

                lax.fori_loop(0, (NBAT - 2) // 2, pipe, 0)
                consume(0)
                consume(1)
                drain(0)
                drain(1)
                plsc.subcore_barrier()

                r0 = sid * dr
                for st in _steps(dr):
                    pltpu.sync_copy(acc.at[pl.ds(r0, st)], rb0.at[pl.ds(0, st)])
                    pltpu.sync_copy(rb0.at[pl.ds(0, st)], s_hbm.at[pl.ds(base + r0, st)])
                    r0 += st
                if with_counts:
                    pltpu.sync_copy(cacc.at[pl.ds(sid * dr, dr)],
                                    cbufb.at[pl.ds(0, dr)])
                    pltpu.sync_copy(cbufb.at[pl.ds(0, dr)],
                                    c_hbm.at[pl.ds(base + sid * dr, dr)])
                    for k in range(768 // 16):
                        cbufb[pl.ds(16 * k, 16)] = zero16
                plsc.subcore_barrier()

    return agg


def _make_combine(n_rows, n_rel):
    R = 128

    def body(*refs):
        x_ref = refs[2 * n_rel]
        wl = refs[2 * n_rel + 1: 2 * n_rel + 1 + n_rel]
        wr = refs[3 * n_rel + 1]
        b = refs[3 * n_rel + 2]
        o = refs[-1]
        acc = jnp.dot(x_ref[...], wr[...], preferred_element_type=F32) + b[...]
        for r in range(n_rel):
            s = refs[2 * r][...]
            c = refs[2 * r + 1][...]
            mean = s / jnp.maximum(c, 1.0)
            acc = acc + jnp.dot(mean, wl[r][...], preferred_element_type=F32)
        o[...] = jnp.maximum(acc, 0.0)

    in_specs = []
    for _ in range(n_rel):
        in_specs.append(pl.BlockSpec((R, D), lambda i: (i, 0)))
        in_specs.append(pl.BlockSpec((R, 1), lambda i: (i, 0)))
    in_specs.append(pl.BlockSpec((R, D), lambda i: (i, 0)))
    for _ in range(n_rel):
        in_specs.append(pl.BlockSpec((D, D), lambda i: (0, 0)))
    in_specs.append(pl.BlockSpec((D, D), lambda i: (0, 0)))
    in_specs.append(pl.BlockSpec((1, D), lambda i: (0, 0)))
    return pl.pallas_call(
        body,
        grid=(n_rows // R,),
        in_specs=in_specs,
        out_specs=pl.BlockSpec((R, D), lambda i: (i, 0)),
        out_shape=jax.ShapeDtypeStruct((n_rows, D), F32),
    )


def _make_head(n_rows):
    R = 128

    def body(hm, hg, lw, lb, gw, gb, l2w, l2b, o):
        logits = jnp.dot(hm[...], lw[...], preferred_element_type=F32) + lb[...]
        ll1 = jnp.dot(hg[...], gw[...], preferred_element_type=F32) + gb[...]
        temp = jnp.dot(ll1, l2w[...], preferred_element_type=F32) + l2b[...]
        o[...] = logits / temp

    in_specs = [
        pl.BlockSpec((R, D), lambda i: (i, 0)),
        pl.BlockSpec((R, D), lambda i: (i, 0)),
        pl.BlockSpec((D, O), lambda i: (0, 0)),
        pl.BlockSpec((1, O), lambda i: (0, 0)),
        pl.BlockSpec((D, O), lambda i: (0, 0)),
        pl.BlockSpec((1, O), lambda i: (0, 0)),
        pl.BlockSpec((O, 1), lambda i: (0, 0)),
        pl.BlockSpec((1, 1), lambda i: (0, 0)),
    ]
    return pl.pallas_call(
        body,
        grid=(n_rows // R,),
        in_specs=in_specs,
        out_specs=pl.BlockSpec((R, O), lambda i: (i, 0)),
        out_shape=jax.ShapeDtypeStruct((n_rows, O), F32),
    )


def kernel(x_author, x_paper, x_term, edge_ap, edge_pa, edge_pt, edge_tp, params):
    xa = jnp.pad(x_author, ((0, NAP - NA), (0, 0)))
    xp = jnp.pad(x_paper, ((0, NPP - NP_), (0, 0)))
    xt = jnp.pad(x_term, ((0, NTP - NT), (0, 0)))

    npad = E_PAD - E_RAW

    def prep_edges(e):
        src = jnp.concatenate([e[0], jnp.arange(npad, dtype=I32) % 997])
        dst = jnp.concatenate([e[1], jnp.full((npad,), -1, I32)])
        return jnp.stack([src, dst])

    eap = prep_edges(edge_ap)
    epa = prep_edges(edge_pa)
    ept = prep_edges(edge_pt)
    etp = prep_edges(edge_tp)

    agg_ap = _make_agg(5, 11648, 0)
    agg_tp = _make_agg(5, 11648, 1)
    agg_pa = _make_agg(2, 6144, 0)
    agg_pt = _make_agg(2, 2560, 0)
    agg_ap_nc = _make_agg(5, 11648, 0, with_counts=False)
    agg_tp_nc = _make_agg(5, 11648, 1, with_counts=False)
    agg_pa_nc = _make_agg(2, 6144, 0, with_counts=False)
    agg_pt_nc = _make_agg(2, 2560, 0, with_counts=False)

    comb_a = _make_combine(NAP, 1)
    comb_p = _make_combine(NPP, 2)
    comb_t = _make_combine(NTP, 1)
    head = _make_head(NAP)

    def aggregate(xd, first=False):
        if first:
            return {
                "ap": agg_ap(eap, xd["author"]),
                "pa": agg_pa(epa, xd["paper"]),
                "pt": agg_pt(ept, xd["paper"]),
                "tp": agg_tp(etp, xd["term"]),
            }
        return {
            "ap": agg_ap_nc(eap, xd["author"]),
            "pa": agg_pa_nc(epa, xd["paper"]),
            "pt": agg_pt_nc(ept, xd["paper"]),
            "tp": agg_tp_nc(etp, xd["term"]),
        }

    def hetero(sd, cd, xd, lp):
        out_a = comb_a(sd["pa"], cd["pa"], xd["author"],
                       lp["pa"]["Wl"], lp["pa"]["Wr"], lp["pa"]["bl"].reshape(1, D))
        out_p = comb_p(sd["ap"], cd["ap"], sd["tp"], cd["tp"], xd["paper"],
                       lp["ap"]["Wl"], lp["tp"]["Wl"],
                       lp["ap"]["Wr"] + lp["tp"]["Wr"],
                       (lp["ap"]["bl"] + lp["tp"]["bl"]).reshape(1, D))
        out_t = comb_t(sd["pt"], cd["pt"], xd["term"],
                       lp["pt"]["Wl"], lp["pt"]["Wr"], lp["pt"]["bl"].reshape(1, D))
        return {"author": out_a, "paper": out_p, "term": out_t}

    m = params["model"]
    g = params["gts"]
    cv = params["convs"]

    xd0 = {"author": xa, "paper": xp, "term": xt}
    p1 = aggregate(xd0, first=True)
    cd = {r: p1[r][1].reshape(-1, 1) for r in p1}
    s1 = {r: p1[r][0] for r in p1}
    h1m = hetero(s1, cd, xd0, m["layers"][0])
    h1c = hetero(s1, cd, xd0, cv[0])
    s2 = aggregate(h1m)
    h2m = hetero(s2, cd, h1m, m["layers"][1])
    s3 = aggregate(h1c)
    h2c = hetero(s3, cd, h1c, cv[1])
    s4 = aggregate(h2m)
    h1g = hetero(s4, cd, h2m, g["layers"][0])
    s5 = aggregate(h1g)
    h2g = hetero(s5, cd, h1g, g["layers"][1])

    out0 = head(h2m["author"], h2g["author"],
                m["lin_W"], m["lin_b"].reshape(1, O),
                g["lin_W"], g["lin_b"].reshape(1, O),
                params["lin2_W"], params["lin2_b"].reshape(1, 1))
    return (out0[:NA], h2c["author"][:NA], h2c["paper"][:NP_], h2c["term"][:NT])
